# Initial kernel scaffold; baseline (speedup 1.0000x reference)
import jax
import jax.numpy as jnp
from jax.experimental import pallas as pl


def kernel(x, W_route, b_route, W_noise, b_noise, W_experts):
    raise NotImplementedError("write your pallas kernel here")


# speedup vs baseline: 2.7182x; 2.7182x over previous
import functools

import jax
import jax.numpy as jnp
from jax.experimental import pallas as pl
from jax.experimental.pallas import tpu as pltpu

N_EXP = 16
BLK = 512


def _fused_kernel(x_ref, w_ref, b_ref, o_ref):
    xb = x_ref[...]
    wc = w_ref[...]
    comb = jnp.dot(xb, wc, preferred_element_type=jnp.float32)
    logits = comb[:, :N_EXP] + b_ref[...]
    experts = comb[:, N_EXP:]

    idx = jax.lax.broadcasted_iota(jnp.int32, logits.shape, 1)
    m1 = jnp.max(logits, axis=1, keepdims=True)
    i1 = jnp.min(jnp.where(logits == m1, idx, N_EXP), axis=1, keepdims=True)
    oh1 = idx == i1
    rest = jnp.where(oh1, -jnp.inf, logits)
    m2 = jnp.max(rest, axis=1, keepdims=True)
    i2 = jnp.min(jnp.where(rest == m2, idx, N_EXP), axis=1, keepdims=True)
    oh2 = idx == i2

    e2 = jnp.exp(m2 - m1)
    denom = 1.0 + e2
    w1 = 1.0 / denom
    w2 = e2 / denom
    e1v = jnp.sum(jnp.where(oh1, experts, 0.0), axis=1, keepdims=True)
    e2v = jnp.sum(jnp.where(oh2, experts, 0.0), axis=1, keepdims=True)
    o_ref[...] = w1 * e1v + w2 * e2v


@functools.partial(jax.jit, static_argnames=())
def kernel(x, W_route, b_route, W_noise, b_noise, W_experts):
    n, d = x.shape
    wc = jnp.concatenate([W_noise, W_experts], axis=0).T
    b2 = b_noise.reshape(1, N_EXP)
    grid = (n // BLK,)
    out = pl.pallas_call(
        _fused_kernel,
        grid=grid,
        in_specs=[
            pl.BlockSpec((BLK, d), lambda i: (i, 0)),
            pl.BlockSpec((d, 2 * N_EXP), lambda i: (0, 0)),
            pl.BlockSpec((1, N_EXP), lambda i: (0, 0)),
        ],
        out_specs=pl.BlockSpec((BLK, 1), lambda i: (i, 0)),
        out_shape=jax.ShapeDtypeStruct((n, 1), jnp.float32),
        compiler_params=pltpu.CompilerParams(
            dimension_semantics=("arbitrary",),
        ),
    )(x, wc, b2)
    return out


# speedup vs baseline: 2.9403x; 1.0817x over previous
import functools

import jax
import jax.numpy as jnp
from jax.experimental import pallas as pl
from jax.experimental.pallas import tpu as pltpu

N_EXP = 16
BLK = 1024


def _fused_kernel(x_ref, w_ref, b_ref, o_ref):
    xb = x_ref[...]
    wc = w_ref[...]
    comb = jnp.dot(xb, wc, preferred_element_type=jnp.float32)
    logits = comb[:, :N_EXP] + b_ref[...]
    experts = comb[:, N_EXP:]

    idx = jax.lax.broadcasted_iota(jnp.int32, logits.shape, 1)
    m1 = jnp.max(logits, axis=1, keepdims=True)
    i1 = jnp.min(jnp.where(logits == m1, idx, N_EXP), axis=1, keepdims=True)
    oh1 = idx == i1
    rest = jnp.where(oh1, -jnp.inf, logits)
    m2 = jnp.max(rest, axis=1, keepdims=True)
    i2 = jnp.min(jnp.where(rest == m2, idx, N_EXP), axis=1, keepdims=True)
    oh2 = idx == i2

    e2 = jnp.exp(m2 - m1)
    denom = 1.0 + e2
    w1 = 1.0 / denom
    w2 = e2 / denom
    e1v = jnp.sum(jnp.where(oh1, experts, 0.0), axis=1, keepdims=True)
    e2v = jnp.sum(jnp.where(oh2, experts, 0.0), axis=1, keepdims=True)
    o_ref[...] = w1 * e1v + w2 * e2v


@functools.partial(jax.jit, static_argnames=())
def kernel(x, W_route, b_route, W_noise, b_noise, W_experts):
    n, d = x.shape
    wc = jnp.concatenate([W_noise, W_experts], axis=0).T
    b2 = b_noise.reshape(1, N_EXP)
    grid = (n // BLK,)
    out = pl.pallas_call(
        _fused_kernel,
        grid=grid,
        in_specs=[
            pl.BlockSpec((BLK, d), lambda i: (i, 0)),
            pl.BlockSpec((d, 2 * N_EXP), lambda i: (0, 0)),
            pl.BlockSpec((1, N_EXP), lambda i: (0, 0)),
        ],
        out_specs=pl.BlockSpec((BLK, 1), lambda i: (i, 0)),
        out_shape=jax.ShapeDtypeStruct((n, 1), jnp.float32),
        compiler_params=pltpu.CompilerParams(
            dimension_semantics=("arbitrary",),
        ),
    )(x, wc, b2)
    return out
